# baseline (speedup 1.0000x reference)
import jax
import jax.numpy as jnp
from jax import lax
from jax.experimental import pallas as pl
from jax.experimental.pallas import tpu as pltpu
from jax.experimental.pallas import tpu_sc as plsc

N = 2_000_000
K = 100_000

NC = 2
NS = 16
NW = NC * NS

KP = 100_352
TBLK = 1024

CH = 2048
IDR = CH // 128
FULLC = N // CH
TAIL = N - FULLC * CH
TAILB = TAIL // 128
TROW = FULLC * CH // 128
PAIRS = 15
EXTRA = FULLC - NW * 2 * PAIRS


def _tc_table_body(rot_ref, trans_ref, out_ref):
    wx, wy, wz = rot_ref[0], rot_ref[1], rot_ref[2]
    tx, ty, tz = trans_ref[0], trans_ref[1], trans_ref[2]
    t2 = wx * wx + wy * wy + wz * wz
    theta = jnp.sqrt(t2)
    inv = 1.0 / (theta + 1e-8)
    kx, ky, kz = wx * inv, wy * inv, wz * inv
    small = theta < 1e-4
    s = jnp.where(small, 1.0, jnp.sin(theta))
    omc = jnp.where(small, 0.0, 1.0 - jnp.cos(theta))
    nk2 = kx * kx + ky * ky + kz * kz
    kxy, kxz, kyz = kx * ky, kx * kz, ky * kz
    zero = jnp.zeros_like(wx)
    comps = [
        1.0 + omc * (kx * kx - nk2),
        -s * kz + omc * kxy,
        s * ky + omc * kxz,
        s * kz + omc * kxy,
        1.0 + omc * (ky * ky - nk2),
        -s * kx + omc * kyz,
        -s * ky + omc * kxz,
        s * kx + omc * kyz,
        1.0 + omc * (kz * kz - nk2),
        tx, ty, tz, zero, zero, zero, zero,
    ]
    for j, v in enumerate(comps):
        out_ref[j] = v


def _sc_transform_body(px_hbm, py_hbm, pz_hbm, ids_hbm, planes_hbm,
                       out_hbm, table_hbm,
                       idx0, idx1, rows0, rows1,
                       px0, py0, pz0, px1, py1, pz1,
                       ox0, oy0, oz0, ox1, oy1, oz1,
                       prep_v, sid0, sid1, sin0, sin1, sg0, sg1, so0, so1):
    wid = lax.axis_index("s") * NC + lax.axis_index("c")
    col_c = [jnp.full((16,), j, jnp.int32) for j in range(12)]
    iota = lax.iota(jnp.int32, 16)

    bufs = [
        dict(idx=idx0, rows=rows0, px=px0, py=py0, pz=pz0,
             ox=ox0, oy=oy0, oz=oz0, sid=sid0, sin=sin0, sg=sg0, so=so0),
        dict(idx=idx1, rows=rows1, px=px1, py=py1, pz=pz1,
             ox=ox1, oy=oy1, oz=oz1, sid=sid1, sin=sin1, sg=sg1, so=so1),
    ]

    def chunk(j):
        return wid + NW * j

    def issue_ids(c, b):
        B = bufs[b]
        pltpu.async_copy(ids_hbm.at[pl.ds(c * IDR, IDR)], B["idx"], B["sid"])

    def wait_ids(b):
        B = bufs[b]
        pltpu.make_async_copy(ids_hbm.at[pl.ds(0, IDR)], B["idx"],
                              B["sid"]).wait()

    def issue_body(c, b):
        B = bufs[b]
        pbase = pl.multiple_of(c * CH, 8)
        pltpu.async_copy(px_hbm.at[pl.ds(pbase, CH)], B["px"], B["sin"])
        pltpu.async_copy(py_hbm.at[pl.ds(pbase, CH)], B["py"], B["sin"])
        pltpu.async_copy(pz_hbm.at[pl.ds(pbase, CH)], B["pz"], B["sin"])
        for s in range(IDR):
            pltpu.async_copy(table_hbm.at[B["idx"].at[s]],
                             B["rows"].at[pl.ds(s * 128, 128)], B["sg"])

    def wait_body(b):
        B = bufs[b]
        pltpu.make_async_copy(px_hbm.at[pl.ds(0, CH)], B["px"], B["sin"]).wait()
        pltpu.make_async_copy(py_hbm.at[pl.ds(0, CH)], B["py"], B["sin"]).wait()
        pltpu.make_async_copy(pz_hbm.at[pl.ds(0, CH)], B["pz"], B["sin"]).wait()
        for s in range(IDR):
            pltpu.make_async_copy(table_hbm.at[pl.ds(0, 128)],
                                  B["rows"].at[pl.ds(s * 128, 128)],
                                  B["sg"]).wait()

    def issue_out(c, b):
        B = bufs[b]
        pbase = pl.multiple_of(c * CH, 8)
        pltpu.async_copy(B["ox"], out_hbm.at[0, pl.ds(pbase, CH)], B["so"])
        pltpu.async_copy(B["oy"], out_hbm.at[1, pl.ds(pbase, CH)], B["so"])
        pltpu.async_copy(B["oz"], out_hbm.at[2, pl.ds(pbase, CH)], B["so"])

    def wait_out(b, n=CH):
        B = bufs[b]
        pltpu.make_async_copy(B["ox"].at[pl.ds(0, n)],
                              out_hbm.at[0, pl.ds(0, n)], B["so"]).wait()
        pltpu.make_async_copy(B["oy"].at[pl.ds(0, n)],
                              out_hbm.at[1, pl.ds(0, n)], B["so"]).wait()
        pltpu.make_async_copy(B["oz"].at[pl.ds(0, n)],
                              out_hbm.at[2, pl.ds(0, n)], B["so"]).wait()

    def compute(b, ngroups):
        B = bufs[b]
        rows_v, pxv, pyv, pzv = B["rows"], B["px"], B["py"], B["pz"]
        oxv, oyv, ozv = B["ox"], B["oy"], B["oz"]

        def group_body(i, carry2):
            ridx = iota + i * 16
            px = pxv[pl.ds(i * 16, 16)]
            py = pyv[pl.ds(i * 16, 16)]
            pz = pzv[pl.ds(i * 16, 16)]
            r00 = plsc.load_gather(rows_v, [ridx, col_c[0]])
            r01 = plsc.load_gather(rows_v, [ridx, col_c[1]])
            r02 = plsc.load_gather(rows_v, [ridx, col_c[2]])
            r10 = plsc.load_gather(rows_v, [ridx, col_c[3]])
            r11 = plsc.load_gather(rows_v, [ridx, col_c[4]])
            r12 = plsc.load_gather(rows_v, [ridx, col_c[5]])
            r20 = plsc.load_gather(rows_v, [ridx, col_c[6]])
            r21 = plsc.load_gather(rows_v, [ridx, col_c[7]])
            r22 = plsc.load_gather(rows_v, [ridx, col_c[8]])
            t0 = plsc.load_gather(rows_v, [ridx, col_c[9]])
            t1 = plsc.load_gather(rows_v, [ridx, col_c[10]])
            t2 = plsc.load_gather(rows_v, [ridx, col_c[11]])
            oxv[pl.ds(i * 16, 16)] = r00 * px + r01 * py + r02 * pz + t0
            oyv[pl.ds(i * 16, 16)] = r10 * px + r11 * py + r12 * pz + t1
            ozv[pl.ds(i * 16, 16)] = r20 * px + r21 * py + r22 * pz + t2
            return carry2
        lax.fori_loop(0, ngroups, group_body, 0)

    sid = lax.axis_index("s")
    PROWS = (KP // 128) // NS
    PSUBR = 7
    PSUB = PSUBR * 128
    for sub in range(PROWS // PSUBR):
        rbase = sid * PROWS + sub * PSUBR
        for j in range(12):
            pltpu.async_copy(planes_hbm.at[pl.ds(j * (KP // 128) + rbase,
                                                 PSUBR)],
                             prep_v.at[j], sg0)
        for j in range(12):
            pltpu.make_async_copy(planes_hbm.at[pl.ds(0, PSUBR)],
                                  prep_v.at[j], sg0).wait()

        def prep_group(i, carry):
            ridx = iota + i * 16
            r = i // 8
            q = (i % 8) * 16
            for j in range(12):
                vj = prep_v[j, r, pl.ds(q, 16)]
                plsc.store_scatter(rows0, [ridx, col_c[j]], vj)
            return carry
        lax.fori_loop(0, PSUB // 16, prep_group, 0)
        pltpu.sync_copy(rows0.at[pl.ds(0, PSUB)],
                        table_hbm.at[pl.ds(rbase * 128, PSUB)])
    plsc.subcore_barrier()

    has_extra = wid < EXTRA

    pltpu.sync_copy(ids_hbm.at[pl.ds(chunk(0) * IDR, IDR)], bufs[0]["idx"])
    issue_body(chunk(0), 0)
    issue_ids(chunk(1), 1)

    def pair_body(jj, carry):
        j0 = 2 * jj
        wait_ids(1)
        issue_body(chunk(j0 + 1), 1)
        wait_body(0)

        @pl.when((jj < PAIRS - 1) | has_extra)
        def _():
            issue_ids(chunk(j0 + 2), 0)

        @pl.when(jj > 0)
        def _():
            wait_out(0)
        compute(0, CH // 16)
        issue_out(chunk(j0), 0)

        @pl.when((jj < PAIRS - 1) | has_extra)
        def _():
            wait_ids(0)
            issue_body(chunk(j0 + 2), 0)
        wait_body(1)

        @pl.when(jj < PAIRS - 1)
        def _():
            issue_ids(chunk(j0 + 3), 1)

        @pl.when(jj > 0)
        def _():
            wait_out(1)
        compute(1, CH // 16)
        issue_out(chunk(j0 + 1), 1)
        return carry

    lax.fori_loop(0, PAIRS, pair_body, 0)

    @pl.when(has_extra)
    def _():
        wait_body(0)
        wait_out(0)
        compute(0, CH // 16)
        issue_out(chunk(2 * PAIRS), 0)

    @pl.when(wid < TAILB)
    def _():
        B = bufs[1]
        tbase = pl.multiple_of(FULLC * CH + wid * 128, 8)
        pltpu.sync_copy(ids_hbm.at[pl.ds(TROW + wid, 1)],
                        B["idx"].at[pl.ds(0, 1)])
        pltpu.sync_copy(px_hbm.at[pl.ds(tbase, 128)], B["px"].at[pl.ds(0, 128)])
        pltpu.sync_copy(py_hbm.at[pl.ds(tbase, 128)], B["py"].at[pl.ds(0, 128)])
        pltpu.sync_copy(pz_hbm.at[pl.ds(tbase, 128)], B["pz"].at[pl.ds(0, 128)])
        pltpu.async_copy(table_hbm.at[B["idx"].at[0]],
                         B["rows"].at[pl.ds(0, 128)], B["sg"]).wait()
        wait_out(1)
        compute(1, 8)
        B2 = bufs[1]
        pltpu.async_copy(B2["ox"].at[pl.ds(0, 128)],
                         out_hbm.at[0, pl.ds(tbase, 128)], B2["so"])
        pltpu.async_copy(B2["oy"].at[pl.ds(0, 128)],
                         out_hbm.at[1, pl.ds(tbase, 128)], B2["so"])
        pltpu.async_copy(B2["oz"].at[pl.ds(0, 128)],
                         out_hbm.at[2, pl.ds(tbase, 128)], B2["so"])

    wait_out(0)

    @pl.when(wid < TAILB)
    def _():
        wait_out(1, n=128)

    @pl.when(wid >= TAILB)
    def _():
        wait_out(1)


def kernel(points, cluster_ids, rotation_params, translation_params):
    rot = jnp.pad(rotation_params, ((0, KP - K), (0, 0)))
    trn = jnp.pad(translation_params, ((0, KP - K), (0, 0)))
    rot3 = rot.T.reshape(3, KP // 128, 128)
    trn3 = trn.T.reshape(3, KP // 128, 128)
    nsteps = KP // TBLK
    planes = pl.pallas_call(
        _tc_table_body,
        grid=(nsteps,),
        in_specs=[
            pl.BlockSpec((3, TBLK // 128, 128), lambda i: (0, i, 0)),
            pl.BlockSpec((3, TBLK // 128, 128), lambda i: (0, i, 0)),
        ],
        out_specs=pl.BlockSpec((16, TBLK // 128, 128), lambda i: (0, i, 0)),
        out_shape=jax.ShapeDtypeStruct((16, KP // 128, 128), jnp.float32),
    )(rot3, trn3)
    planes2d = planes.reshape(16 * (KP // 128), 128)

    px = points[:, 0]
    py = points[:, 1]
    pz = points[:, 2]
    ids2d = cluster_ids.astype(jnp.int32).reshape(N // 128, 128)

    mesh = plsc.VectorSubcoreMesh(core_axis_name="c", subcore_axis_name="s",
                                  num_cores=NC, num_subcores=NS)
    fn = pl.kernel(
        _sc_transform_body,
        out_type=(jax.ShapeDtypeStruct((3, N), jnp.float32),
                  jax.ShapeDtypeStruct((KP, 16), jnp.float32)),
        mesh=mesh,
        compiler_params=pltpu.CompilerParams(needs_layout_passes=False,
                                             use_tc_tiling_on_sc=False),
        scratch_types=(
            [pltpu.VMEM((IDR, 128), jnp.int32)] * 2
            + [pltpu.VMEM((CH, 16), jnp.float32)] * 2
            + [pltpu.VMEM((CH,), jnp.float32)] * 12
            + [pltpu.VMEM((12, 7, 128), jnp.float32)]
            + [pltpu.SemaphoreType.DMA] * 8
        ),
    )
    out3, _ = fn(px, py, pz, ids2d, planes2d)
    return out3.T


# speedup vs baseline: 2.0596x; 2.0596x over previous
import jax
import jax.numpy as jnp
from jax import lax
from jax.experimental import pallas as pl
from jax.experimental.pallas import tpu as pltpu
from jax.experimental.pallas import tpu_sc as plsc

N = 2_000_000
K = 100_000

NC = 2
NS = 16
NW = NC * NS

KP = 100_352
TBLK = 1024

CH = 2048
IDR = CH // 128
FULLC = N // CH
TAIL = N - FULLC * CH
TAILB = TAIL // 128
TROW = FULLC * CH // 128
PAIRS = 15
EXTRA = FULLC - NW * 2 * PAIRS


def _tc_table_body(rot_ref, trans_ref, out_ref):
    wx, wy, wz = rot_ref[0], rot_ref[1], rot_ref[2]
    tx, ty, tz = trans_ref[0], trans_ref[1], trans_ref[2]
    t2 = wx * wx + wy * wy + wz * wz
    theta = jnp.sqrt(t2)
    inv = 1.0 / (theta + 1e-8)
    kx, ky, kz = wx * inv, wy * inv, wz * inv
    small = theta < 1e-4
    s = jnp.where(small, 1.0, jnp.sin(theta))
    omc = jnp.where(small, 0.0, 1.0 - jnp.cos(theta))
    nk2 = kx * kx + ky * ky + kz * kz
    kxy, kxz, kyz = kx * ky, kx * kz, ky * kz
    zero = jnp.zeros_like(wx)
    comps = [
        1.0 + omc * (kx * kx - nk2),
        -s * kz + omc * kxy,
        s * ky + omc * kxz,
        s * kz + omc * kxy,
        1.0 + omc * (ky * ky - nk2),
        -s * kx + omc * kyz,
        -s * ky + omc * kxz,
        s * kx + omc * kyz,
        1.0 + omc * (kz * kz - nk2),
        tx, ty, tz, zero, zero, zero, zero,
    ]
    for j, v in enumerate(comps):
        out_ref[j] = v


def _sc_transform_body(px_hbm, py_hbm, pz_hbm, ids_hbm, planes_hbm,
                       ox_hbm, oy_hbm, oz_hbm, table_hbm,
                       idx0, idx1, rows0, rows1,
                       px0, py0, pz0, px1, py1, pz1,
                       ox0, oy0, oz0, ox1, oy1, oz1,
                       prep_v, sid0, sid1, sin0, sin1, sg0, sg1, so0, so1):
    wid = lax.axis_index("s") * NC + lax.axis_index("c")
    col_c = [jnp.full((16,), j, jnp.int32) for j in range(12)]
    iota = lax.iota(jnp.int32, 16)

    bufs = [
        dict(idx=idx0, rows=rows0, px=px0, py=py0, pz=pz0,
             ox=ox0, oy=oy0, oz=oz0, sid=sid0, sin=sin0, sg=sg0, so=so0),
        dict(idx=idx1, rows=rows1, px=px1, py=py1, pz=pz1,
             ox=ox1, oy=oy1, oz=oz1, sid=sid1, sin=sin1, sg=sg1, so=so1),
    ]

    def chunk(j):
        return wid + NW * j

    def issue_ids(c, b):
        B = bufs[b]
        pltpu.async_copy(ids_hbm.at[pl.ds(c * IDR, IDR)], B["idx"], B["sid"])

    def wait_ids(b):
        B = bufs[b]
        pltpu.make_async_copy(ids_hbm.at[pl.ds(0, IDR)], B["idx"],
                              B["sid"]).wait()

    def issue_planes(c, b):
        B = bufs[b]
        pbase = pl.multiple_of(c * CH, 8)
        pltpu.async_copy(px_hbm.at[pl.ds(pbase, CH)], B["px"], B["sin"])
        pltpu.async_copy(py_hbm.at[pl.ds(pbase, CH)], B["py"], B["sin"])
        pltpu.async_copy(pz_hbm.at[pl.ds(pbase, CH)], B["pz"], B["sin"])

    def issue_gathers(b):
        B = bufs[b]
        for s in range(IDR):
            pltpu.async_copy(table_hbm.at[B["idx"].at[s]],
                             B["rows"].at[pl.ds(s * 128, 128)], B["sg"])

    def issue_body(c, b):
        issue_planes(c, b)
        issue_gathers(b)

    def wait_body(b):
        B = bufs[b]
        pltpu.make_async_copy(px_hbm.at[pl.ds(0, CH)], B["px"], B["sin"]).wait()
        pltpu.make_async_copy(py_hbm.at[pl.ds(0, CH)], B["py"], B["sin"]).wait()
        pltpu.make_async_copy(pz_hbm.at[pl.ds(0, CH)], B["pz"], B["sin"]).wait()
        for s in range(IDR):
            pltpu.make_async_copy(table_hbm.at[pl.ds(0, 128)],
                                  B["rows"].at[pl.ds(s * 128, 128)],
                                  B["sg"]).wait()

    def issue_out(c, b):
        B = bufs[b]
        pbase = pl.multiple_of(c * CH, 8)
        pltpu.async_copy(B["ox"], ox_hbm.at[pl.ds(pbase, CH)], B["so"])
        pltpu.async_copy(B["oy"], oy_hbm.at[pl.ds(pbase, CH)], B["so"])
        pltpu.async_copy(B["oz"], oz_hbm.at[pl.ds(pbase, CH)], B["so"])

    def wait_out(b, n=CH):
        B = bufs[b]
        pltpu.make_async_copy(B["ox"].at[pl.ds(0, n)],
                              ox_hbm.at[pl.ds(0, n)], B["so"]).wait()
        pltpu.make_async_copy(B["oy"].at[pl.ds(0, n)],
                              oy_hbm.at[pl.ds(0, n)], B["so"]).wait()
        pltpu.make_async_copy(B["oz"].at[pl.ds(0, n)],
                              oz_hbm.at[pl.ds(0, n)], B["so"]).wait()

    def compute(b, ngroups):
        B = bufs[b]
        rows_v, pxv, pyv, pzv = B["rows"], B["px"], B["py"], B["pz"]
        oxv, oyv, ozv = B["ox"], B["oy"], B["oz"]

        def group_body(i, carry2):
            ridx = iota + i * 16
            px = pxv[pl.ds(i * 16, 16)]
            py = pyv[pl.ds(i * 16, 16)]
            pz = pzv[pl.ds(i * 16, 16)]
            r00 = plsc.load_gather(rows_v, [ridx, col_c[0]])
            r01 = plsc.load_gather(rows_v, [ridx, col_c[1]])
            r02 = plsc.load_gather(rows_v, [ridx, col_c[2]])
            r10 = plsc.load_gather(rows_v, [ridx, col_c[3]])
            r11 = plsc.load_gather(rows_v, [ridx, col_c[4]])
            r12 = plsc.load_gather(rows_v, [ridx, col_c[5]])
            r20 = plsc.load_gather(rows_v, [ridx, col_c[6]])
            r21 = plsc.load_gather(rows_v, [ridx, col_c[7]])
            r22 = plsc.load_gather(rows_v, [ridx, col_c[8]])
            t0 = plsc.load_gather(rows_v, [ridx, col_c[9]])
            t1 = plsc.load_gather(rows_v, [ridx, col_c[10]])
            t2 = plsc.load_gather(rows_v, [ridx, col_c[11]])
            oxv[pl.ds(i * 16, 16)] = r00 * px + r01 * py + r02 * pz + t0
            oyv[pl.ds(i * 16, 16)] = r10 * px + r11 * py + r12 * pz + t1
            ozv[pl.ds(i * 16, 16)] = r20 * px + r21 * py + r22 * pz + t2
            return carry2
        lax.fori_loop(0, ngroups, group_body, 0)

    issue_ids(chunk(0), 0)
    issue_planes(chunk(0), 0)
    issue_ids(chunk(1), 1)

    sid = lax.axis_index("s")
    PROWS = (KP // 128) // NS
    PSUBR = 7
    PSUB = PSUBR * 128
    for sub in range(PROWS // PSUBR):
        rbase = sid * PROWS + sub * PSUBR
        for j in range(12):
            pltpu.async_copy(planes_hbm.at[pl.ds(j * (KP // 128) + rbase,
                                                 PSUBR)],
                             prep_v.at[j], sg0)
        for j in range(12):
            pltpu.make_async_copy(planes_hbm.at[pl.ds(0, PSUBR)],
                                  prep_v.at[j], sg0).wait()

        def prep_group(i, carry):
            ridx = iota + i * 16
            r = i // 8
            q = (i % 8) * 16
            for j in range(12):
                vj = prep_v[j, r, pl.ds(q, 16)]
                plsc.store_scatter(rows0, [ridx, col_c[j]], vj)
            return carry
        lax.fori_loop(0, PSUB // 16, prep_group, 0)
        pltpu.sync_copy(rows0.at[pl.ds(0, PSUB)],
                        table_hbm.at[pl.ds(rbase * 128, PSUB)])
    plsc.subcore_barrier()

    has_extra = wid < EXTRA

    wait_ids(0)
    issue_gathers(0)

    def pair_body(jj, carry):
        j0 = 2 * jj
        wait_ids(1)
        issue_body(chunk(j0 + 1), 1)
        wait_body(0)

        @pl.when((jj < PAIRS - 1) | has_extra)
        def _():
            issue_ids(chunk(j0 + 2), 0)

        @pl.when(jj > 0)
        def _():
            wait_out(0)
        compute(0, CH // 16)
        issue_out(chunk(j0), 0)

        @pl.when((jj < PAIRS - 1) | has_extra)
        def _():
            wait_ids(0)
            issue_body(chunk(j0 + 2), 0)
        wait_body(1)

        @pl.when(jj < PAIRS - 1)
        def _():
            issue_ids(chunk(j0 + 3), 1)

        @pl.when(jj > 0)
        def _():
            wait_out(1)
        compute(1, CH // 16)
        issue_out(chunk(j0 + 1), 1)
        return carry

    lax.fori_loop(0, PAIRS, pair_body, 0)

    @pl.when(has_extra)
    def _():
        wait_body(0)
        wait_out(0)
        compute(0, CH // 16)
        issue_out(chunk(2 * PAIRS), 0)

    @pl.when(wid < TAILB)
    def _():
        B = bufs[1]
        tbase = pl.multiple_of(FULLC * CH + wid * 128, 8)
        pltpu.sync_copy(ids_hbm.at[pl.ds(TROW + wid, 1)],
                        B["idx"].at[pl.ds(0, 1)])
        pltpu.sync_copy(px_hbm.at[pl.ds(tbase, 128)], B["px"].at[pl.ds(0, 128)])
        pltpu.sync_copy(py_hbm.at[pl.ds(tbase, 128)], B["py"].at[pl.ds(0, 128)])
        pltpu.sync_copy(pz_hbm.at[pl.ds(tbase, 128)], B["pz"].at[pl.ds(0, 128)])
        pltpu.async_copy(table_hbm.at[B["idx"].at[0]],
                         B["rows"].at[pl.ds(0, 128)], B["sg"]).wait()
        wait_out(1)
        compute(1, 8)
        B2 = bufs[1]
        pltpu.async_copy(B2["ox"].at[pl.ds(0, 128)],
                         ox_hbm.at[pl.ds(tbase, 128)], B2["so"])
        pltpu.async_copy(B2["oy"].at[pl.ds(0, 128)],
                         oy_hbm.at[pl.ds(tbase, 128)], B2["so"])
        pltpu.async_copy(B2["oz"].at[pl.ds(0, 128)],
                         oz_hbm.at[pl.ds(tbase, 128)], B2["so"])

    wait_out(0)

    @pl.when(wid < TAILB)
    def _():
        wait_out(1, n=128)

    @pl.when(wid >= TAILB)
    def _():
        wait_out(1)


def kernel(points, cluster_ids, rotation_params, translation_params):
    rot = jnp.pad(rotation_params, ((0, KP - K), (0, 0)))
    trn = jnp.pad(translation_params, ((0, KP - K), (0, 0)))
    rot3 = rot.T.reshape(3, KP // 128, 128)
    trn3 = trn.T.reshape(3, KP // 128, 128)
    nsteps = KP // TBLK
    planes = pl.pallas_call(
        _tc_table_body,
        grid=(nsteps,),
        in_specs=[
            pl.BlockSpec((3, TBLK // 128, 128), lambda i: (0, i, 0)),
            pl.BlockSpec((3, TBLK // 128, 128), lambda i: (0, i, 0)),
        ],
        out_specs=pl.BlockSpec((16, TBLK // 128, 128), lambda i: (0, i, 0)),
        out_shape=jax.ShapeDtypeStruct((16, KP // 128, 128), jnp.float32),
    )(rot3, trn3)
    planes2d = planes.reshape(16 * (KP // 128), 128)

    px = points[:, 0]
    py = points[:, 1]
    pz = points[:, 2]
    ids2d = cluster_ids.astype(jnp.int32).reshape(N // 128, 128)

    mesh = plsc.VectorSubcoreMesh(core_axis_name="c", subcore_axis_name="s",
                                  num_cores=NC, num_subcores=NS)
    plane = jax.ShapeDtypeStruct((N,), jnp.float32)
    fn = pl.kernel(
        _sc_transform_body,
        out_type=(plane, plane, plane,
                  jax.ShapeDtypeStruct((KP, 16), jnp.float32)),
        mesh=mesh,
        compiler_params=pltpu.CompilerParams(needs_layout_passes=False,
                                             use_tc_tiling_on_sc=False),
        scratch_types=(
            [pltpu.VMEM((IDR, 128), jnp.int32)] * 2
            + [pltpu.VMEM((CH, 16), jnp.float32)] * 2
            + [pltpu.VMEM((CH,), jnp.float32)] * 12
            + [pltpu.VMEM((12, 7, 128), jnp.float32)]
            + [pltpu.SemaphoreType.DMA] * 8
        ),
    )
    ox, oy, oz, _ = fn(px, py, pz, ids2d, planes2d)
    return jnp.stack([ox, oy, oz], axis=1)
